# baseline (speedup 1.0000x reference)
import jax
import jax.numpy as jnp
from jax.experimental import pallas as pl

_BM = 1000


def _fused_heads(x_ref, wc_ref, bc_ref, wb_ref, bb_ref, sc_ref, pd_ref):
    x = x_ref[...]
    sc_ref[...] = (
        jnp.dot(x, wc_ref[...], preferred_element_type=jnp.float32) + bc_ref[...]
    )
    pd_ref[...] = (
        jnp.dot(x, wb_ref[...], preferred_element_type=jnp.float32) + bb_ref[...]
    )


def kernel(x, Wc, bc, Wb, bb):
    if x.ndim > 2:
        x = x.reshape(x.shape[0], -1)
    n, k = x.shape
    wc_t = Wc.T
    wb_t = Wb.T
    nc = wc_t.shape[1]
    nb = wb_t.shape[1]
    scores, deltas = pl.pallas_call(
        _fused_heads,
        grid=(pl.cdiv(n, _BM),),
        in_specs=[
            pl.BlockSpec((_BM, k), lambda i: (i, 0)),
            pl.BlockSpec((k, nc), lambda i: (0, 0)),
            pl.BlockSpec((1, nc), lambda i: (0, 0)),
            pl.BlockSpec((k, nb), lambda i: (0, 0)),
            pl.BlockSpec((1, nb), lambda i: (0, 0)),
        ],
        out_specs=[
            pl.BlockSpec((_BM, nc), lambda i: (i, 0)),
            pl.BlockSpec((_BM, nb), lambda i: (i, 0)),
        ],
        out_shape=[
            jax.ShapeDtypeStruct((n, nc), x.dtype),
            jax.ShapeDtypeStruct((n, nb), x.dtype),
        ],
    )(x, wc_t, bc.reshape(1, nc), wb_t, bb.reshape(1, nb))
    return (scores, deltas)
